# baseline (speedup 1.0000x reference)
import functools

import jax
import jax.numpy as jnp
from jax import lax
from jax.experimental import pallas as pl
from jax.experimental.pallas import tpu as pltpu
from jax.experimental.pallas import tpu_sc as plsc

N = 16384
K = 16
BS = 1024
LANES = 16
NWORKERS = 32
QPW = N // NWORKERS
NBLK = N // LANES
FLUSH = 128
UNROLL = 4
P1_UNROLL = 8
GRIDC = ((0.05, 0.05, 0.05), (0.2, 0.2, 0.2), (0.1, 0.1, 2.0))

_f32 = jnp.float32
_INF = float("inf")


def _bf16r(v):
    u = plsc.bitcast(v, jnp.int32)
    r = u + 0x7FFF + ((u >> 16) & 1)
    r = r & jnp.int32(-65536)
    return plsc.bitcast(r, _f32)


def _sc_knn_body(x_hbm, y_hbm, z_hbm, b_hbm, ox_hbm, oy_hbm, oz_hbm,
                 xv, yv, zv, sxv, xbv, ybv, zbv, probev, bufx, bufy, bufz):
    cid = lax.axis_index("c")
    sid = lax.axis_index("s")
    wid = sid * 2 + cid
    pltpu.sync_copy(x_hbm, xv)
    pltpu.sync_copy(y_hbm, yv)
    pltpu.sync_copy(z_hbm, zv)

    lane_iota = lax.iota(jnp.int32, LANES)

    def _sx_step(b, _):
        s = b * LANES
        cx = xv[pl.ds(s, LANES)]
        cy = yv[pl.ds(s, LANES)]
        cz = zv[pl.ds(s, LANES)]
        sxv[pl.ds(s, LANES)] = cx * cx + cy * cy + cz * cz
        xbv[pl.ds(s, LANES)] = _bf16r(cx)
        ybv[pl.ds(s, LANES)] = _bf16r(cy)
        zbv[pl.ds(s, LANES)] = _bf16r(cz)
        return 0

    lax.fori_loop(0, NBLK, _sx_step, 0)

    def _bs_step(i, lo):
        step = 512 >> i
        nlo = lo + step
        addr = jnp.minimum(nlo, NBLK - 1) * LANES
        pltpu.sync_copy(b_hbm.at[pl.ds(addr, LANES)], probev)
        p0 = probev[...][0]
        return jnp.where((nlo < NBLK) & (p0 == 0), nlo, lo)

    lo_blk = lax.fori_loop(0, 10, _bs_step, jnp.zeros((), jnp.int32))
    pltpu.sync_copy(b_hbm.at[pl.ds(lo_blk * LANES, LANES)], probev)
    bvec = probev[...]
    nzeros = jnp.sum(jnp.where(bvec == 0, 1.0, 0.0)).astype(jnp.int32)
    count0 = lo_blk * LANES + nzeros

    qbase = wid * QPW

    def per_query(qi, k):
        q = qbase + qi
        in0 = q < count0
        seg_lo = jnp.where(in0, 0, count0)
        seg_hi = jnp.where(in0, count0, N)
        chunk = (q - seg_lo) // BS
        ex_lo = seg_lo + chunk * BS
        ex_hi = jnp.minimum(ex_lo + BS, seg_hi)
        lane = qi % LANES
        qblk = qbase + (qi - lane)
        sel = lane_iota == lane
        qsx = jnp.sum(jnp.where(sel, sxv[pl.ds(qblk, LANES)], 0.0))
        qxb = jnp.full((LANES,), jnp.sum(jnp.where(sel, xbv[pl.ds(qblk, LANES)], 0.0)), _f32)
        qyb = jnp.full((LANES,), jnp.sum(jnp.where(sel, ybv[pl.ds(qblk, LANES)], 0.0)), _f32)
        qzb = jnp.full((LANES,), jnp.sum(jnp.where(sel, zbv[pl.ds(qblk, LANES)], 0.0)), _f32)

        def block_key(base):
            prod = (xbv[pl.ds(base, LANES)] * qxb
                    + ybv[pl.ds(base, LANES)] * qyb
                    + zbv[pl.ds(base, LANES)] * qzb)
            t1 = qsx + sxv[pl.ds(base, LANES)]
            return jnp.maximum(t1 - 2.0 * prod, 1e-12)

        def merge16(args):
            td, ti, thr, d2c, b0 = args
            sd, si = plsc.sort_key_val(d2c, b0 + lane_iota)
            rsd = lax.rev(sd, (0,))
            rsi = lax.rev(si, (0,))
            keep = td <= rsd
            md = jnp.where(keep, td, rsd)
            mi = jnp.where(keep, ti, rsi)
            nd, ni = plsc.sort_key_val(md, mi)
            return nd, ni, jnp.minimum(thr, jnp.max(nd))

        def maybe_merge(top_d, top_i, thr, d2m, base):
            bmin = jnp.min(d2m)
            return lax.cond(bmin < thr, merge16,
                            lambda a: (a[0], a[1], a[2]),
                            (top_d, top_i, thr, d2m, base))

        def walk_range(lo, hi, carry, on_masked, on_group, on_block, unroll):
            def do(c):
                b_first = lo // LANES
                b_last = (hi - 1) // LANES
                c = on_masked(b_first, lo, hi, c)
                n_int = jnp.maximum(b_last - b_first - 1, 0)
                n_g = n_int // unroll
                b0 = b_first + 1
                c = lax.fori_loop(
                    0, n_g,
                    lambda g, c2: on_group((b0 + g * unroll) * LANES, c2), c)
                c = lax.fori_loop(
                    0, n_int - n_g * unroll,
                    lambda r, c2: on_block((b0 + n_g * unroll + r) * LANES,
                                           c2), c)
                return lax.cond(b_last > b_first,
                                lambda c2: on_masked(b_last, lo, hi, c2),
                                lambda c2: c2, c)

            return lax.cond(hi > lo, do, lambda c: c, carry)

        def p1_acc(c, d):
            m1, m2 = c
            return (jnp.minimum(m1, d),
                    jnp.minimum(m2, jnp.maximum(m1, d)))

        def p1_masked(b, lo, hi, c):
            base = b * LANES
            gidx = base + lane_iota
            ok = (gidx >= lo) & (gidx < hi)
            return p1_acc(c, jnp.where(ok, block_key(base), _INF))

        def p1_group(gb, c):
            for u in range(P1_UNROLL):
                c = p1_acc(c, block_key(gb + u * LANES))
            return c

        def p1_block(base, c):
            return p1_acc(c, block_key(base))

        m0 = (jnp.full((LANES,), _INF, _f32), jnp.full((LANES,), _INF, _f32))
        m0 = walk_range(seg_lo, ex_lo, m0, p1_masked, p1_group, p1_block,
                        P1_UNROLL)
        m1, m2 = walk_range(ex_hi, seg_hi, m0, p1_masked, p1_group, p1_block,
                            P1_UNROLL)
        s1, _ = plsc.sort_key_val(m1, lane_iota)
        s2, _ = plsc.sort_key_val(m2, lane_iota)
        low16 = jnp.minimum(s1, lax.rev(s2, (0,)))
        thr0 = jnp.max(low16) * (1.0 + 3e-7)

        def p2_masked(b, lo, hi, carry):
            top_d, top_i, thr = carry
            base = b * LANES
            gidx = base + lane_iota
            ok = (gidx >= lo) & (gidx < hi)
            d2m = jnp.where(ok, block_key(base), _INF)
            return maybe_merge(top_d, top_i, thr, d2m, base)

        def p2_group(gb, c2):
            td, ti, thr = c2
            thrv = jnp.full((LANES,), thr, _f32)
            ds = [block_key(gb + u * LANES) for u in range(UNROLL)]
            bits = None
            for u in range(UNROLL):
                nh = plsc.all_reduce_population_count(ds[u] < thrv)
                bu = jnp.where(nh > 0, jnp.int32(1 << u), jnp.int32(0))
                bits = bu if bits is None else bits | bu
            bitsc = bits[0]

            def hit(args):
                t = args
                for u in range(UNROLL):
                    t = lax.cond(
                        ((bitsc >> u) & 1) != 0, merge16,
                        lambda a: (a[0], a[1], a[2]),
                        (*t, ds[u], gb + u * LANES))
                return t

            return lax.cond(bitsc != 0, hit, lambda a: a, (td, ti, thr))

        def p2_block(base, c2):
            td, ti, thr = c2
            return maybe_merge(td, ti, thr, block_key(base), base)

        top = (jnp.full((LANES,), _INF, _f32), lane_iota, thr0)
        top = walk_range(seg_lo, ex_lo, top, p2_masked, p2_group, p2_block,
                         UNROLL)
        top_d, top_i, _ = walk_range(ex_hi, seg_hi, top, p2_masked, p2_group,
                                     p2_block, UNROLL)

        bufx[k, :] = plsc.load_gather(xv, [top_i])
        bufy[k, :] = plsc.load_gather(yv, [top_i])
        bufz[k, :] = plsc.load_gather(zv, [top_i])
        return 0

    def super_group(sg, _):
        def pq(k, _2):
            return per_query(sg * FLUSH + k, k)

        lax.fori_loop(0, FLUSH, pq, 0)
        row0 = qbase + sg * FLUSH
        pltpu.sync_copy(bufx, ox_hbm.at[pl.ds(row0, FLUSH), :])
        pltpu.sync_copy(bufy, oy_hbm.at[pl.ds(row0, FLUSH), :])
        pltpu.sync_copy(bufz, oz_hbm.at[pl.ds(row0, FLUSH), :])
        return 0

    lax.fori_loop(0, QPW // FLUSH, super_group, 0)


@functools.lru_cache(maxsize=1)
def _get_sc_knn():
    return pl.kernel(
        _sc_knn_body,
        out_type=(jax.ShapeDtypeStruct((N, K), _f32),
                  jax.ShapeDtypeStruct((N, K), _f32),
                  jax.ShapeDtypeStruct((N, K), _f32)),
        mesh=plsc.VectorSubcoreMesh(core_axis_name="c", subcore_axis_name="s"),
        scratch_types=[
            pltpu.VMEM((N,), _f32),
            pltpu.VMEM((N,), _f32),
            pltpu.VMEM((N,), _f32),
            pltpu.VMEM((N,), _f32),
            pltpu.VMEM((N,), _f32),
            pltpu.VMEM((N,), _f32),
            pltpu.VMEM((N,), _f32),
            pltpu.VMEM((LANES,), jnp.int32),
            pltpu.VMEM((FLUSH, K), _f32),
            pltpu.VMEM((FLUSH, K), _f32),
            pltpu.VMEM((FLUSH, K), _f32),
        ],
        compiler_params=pltpu.CompilerParams(needs_layout_passes=False,
                                             use_tc_tiling_on_sc=False),
    )


ROWS = 1024
NROWBLK = N // ROWS


def _tc_a_body(feat_ref, w1t_ref, b_ref, h_ref, sums_ref):
    pid = pl.program_id(0)
    h = jnp.dot(feat_ref[...], w1t_ref[...], preferred_element_type=_f32)
    h_ref[...] = h
    m0 = (b_ref[...] == 0).astype(_f32)
    hm0 = h * m0
    hm1 = h - hm0
    s0 = jnp.sum(hm0, axis=0, keepdims=True)
    s0q = jnp.sum(h * hm0, axis=0, keepdims=True)
    s1 = jnp.sum(hm1, axis=0, keepdims=True)
    s1q = jnp.sum(h * hm1, axis=0, keepdims=True)
    c0 = jnp.sum(m0)
    c0row = jnp.full((1, 64), c0, _f32)
    contrib = jnp.concatenate(
        [s0, s0q, s1, s1q, c0row, jnp.zeros((3, 64), _f32)], axis=0)

    @pl.when(pid == 0)
    def _():
        sums_ref[...] = jnp.zeros_like(sums_ref)

    sums_ref[...] += contrib


def _tc_a(feat, w1t, b2d):
    return pl.pallas_call(
        _tc_a_body,
        grid=(NROWBLK,),
        in_specs=[
            pl.BlockSpec((ROWS, 128), lambda i: (i, 0)),
            pl.BlockSpec((128, 64), lambda i: (0, 0)),
            pl.BlockSpec((ROWS, 1), lambda i: (i, 0)),
        ],
        out_specs=[
            pl.BlockSpec((ROWS, 64), lambda i: (i, 0)),
            pl.BlockSpec((8, 64), lambda i: (0, 0)),
        ],
        out_shape=[
            jax.ShapeDtypeStruct((N, 64), _f32),
            jax.ShapeDtypeStruct((8, 64), _f32),
        ],
    )(feat, w1t, b2d)


def _tc_b_body(h_ref, nx_ref, ny_ref, nz_ref, xc_ref, yc_ref, zc_ref,
               sums_ref, aux_ref, w2p_ref, out_ref):
    pid = pl.program_id(0)
    kf = _f32(K)
    nx = nx_ref[...]
    ny = ny_ref[...]
    nz = nz_ref[...]
    xc = xc_ref[...]
    yc = yc_ref[...]
    zc = zc_ref[...]
    dxn = nx - xc
    dyn = ny - yc
    dzn = nz - zc
    def _bf(v):
        return v.astype(jnp.bfloat16).astype(_f32)

    sq = xc * xc + yc * yc + zc * zc
    sn = nx * nx + ny * ny + nz * nz
    prod = _bf(nx) * _bf(xc) + _bf(ny) * _bf(yc) + _bf(nz) * _bf(zc)
    nd2 = (sq + sn) - 2.0 * prod
    nd = jnp.sqrt(jnp.maximum(nd2, 1e-12))
    mean_dist = jnp.mean(nd, axis=1, keepdims=True)
    density = 1.0 / (mean_dist + 1e-6)

    mx = jnp.mean(dxn, axis=1, keepdims=True)
    my = jnp.mean(dyn, axis=1, keepdims=True)
    mz = jnp.mean(dzn, axis=1, keepdims=True)
    inv = _f32(1.0 / (K - 1))
    cxx = (jnp.sum(dxn * dxn, 1, keepdims=True) - kf * mx * mx) * inv
    cyy = (jnp.sum(dyn * dyn, 1, keepdims=True) - kf * my * my) * inv
    czz = (jnp.sum(dzn * dzn, 1, keepdims=True) - kf * mz * mz) * inv
    cxy = (jnp.sum(dxn * dyn, 1, keepdims=True) - kf * mx * my) * inv
    cxz = (jnp.sum(dxn * dzn, 1, keepdims=True) - kf * mx * mz) * inv
    cyz = (jnp.sum(dyn * dzn, 1, keepdims=True) - kf * my * mz) * inv
    q3 = (cxx + cyy + czz) * _f32(1.0 / 3.0)
    bxx = cxx - q3
    byy = cyy - q3
    bzz = czz - q3
    p2 = ((bxx * bxx + byy * byy + bzz * bzz) * _f32(1.0 / 6.0)
          + (cxy * cxy + cxz * cxz + cyz * cyz) * _f32(1.0 / 3.0))
    detb = (bxx * (byy * bzz - cyz * cyz) - cxy * (cxy * bzz - cyz * cxz)
            + cxz * (cxy * cyz - byy * cxz))
    p = jnp.sqrt(p2)
    mu = 2.0 * p
    for _ in range(40):
        g = mu * mu * mu - 3.0 * p2 * mu - detb
        gp = 3.0 * mu * mu - 3.0 * p2
        mu = mu - g / (gp + 1e-30)
    mu = jnp.where(p2 > 1e-30, mu, 0.0)
    linearity = (2.0 * mu - q3) / (3.0 * q3 + 1e-6)

    c0 = sums_ref[4, 0]
    c1 = _f32(N) - c0
    mu0 = sums_ref[0:1, :] / c0
    mu1 = sums_ref[2:3, :] / c1
    v0 = jnp.maximum(sums_ref[1:2, :] / c0 - mu0 * mu0, 0.0)
    v1 = jnp.maximum(sums_ref[3:4, :] / c1 - mu1 * mu1, 0.0)
    rows = lax.broadcasted_iota(jnp.int32, (ROWS, 1), 0) + pid * ROWS
    in0 = rows < c0.astype(jnp.int32)
    h = h_ref[...]
    mean = jnp.where(in0, mu0, mu1)
    var = jnp.where(in0, v0, v1)
    gamma = aux_ref[0:1, :]
    beta = aux_ref[1:2, :]
    hn = (h - mean) / jnp.sqrt(var + 1e-5) * gamma + beta
    h2 = jnp.maximum(hn, 0.0)
    h2b = _bf(h2)
    l0 = jnp.sum(h2b * _bf(w2p_ref[0:1, :]), axis=1, keepdims=True) + aux_ref[2, 0]
    l1 = jnp.sum(h2b * _bf(w2p_ref[1:2, :]), axis=1, keepdims=True) + aux_ref[2, 1]
    l2 = jnp.sum(h2b * _bf(w2p_ref[2:3, :]), axis=1, keepdims=True) + aux_ref[2, 2]
    lm = jnp.maximum(l0, jnp.maximum(l1, l2))
    e0 = jnp.exp(l0 - lm)
    e1 = jnp.exp(l1 - lm)
    e2 = jnp.exp(l2 - lm)
    zs = e0 + e1 + e2
    p0 = e0 / zs
    p1 = e1 / zs
    p2_ = e2 / zs

    tower = (density * 2.0 + p0) * _f32(1.0 / 3.0)
    background = (jnp.maximum(1.0 - linearity, 1.0 - density) + p1) * _f32(1.0 / 3.0)
    line = (linearity * 2.0 + p2_) * _f32(1.0 / 3.0)
    cols = []
    for a in range(3):
        v = (tower * GRIDC[0][a] + background * GRIDC[1][a]
             + line * GRIDC[2][a] + 1e-6)
        cols.append(v)
    computed = jnp.concatenate(cols, axis=1)
    cnt = jnp.where(in0, c0, c1)
    fallback = jnp.concatenate(
        [jnp.full((ROWS, 1), GRIDC[1][a], _f32) for a in range(3)], axis=1)
    out_ref[...] = jnp.where(cnt >= kf, computed, fallback)


def _tc_b(h, nx, ny, nz, xc, yc, zc, sums, aux, w2p):
    return pl.pallas_call(
        _tc_b_body,
        grid=(NROWBLK,),
        in_specs=[
            pl.BlockSpec((ROWS, 64), lambda i: (i, 0)),
            pl.BlockSpec((ROWS, K), lambda i: (i, 0)),
            pl.BlockSpec((ROWS, K), lambda i: (i, 0)),
            pl.BlockSpec((ROWS, K), lambda i: (i, 0)),
            pl.BlockSpec((ROWS, 1), lambda i: (i, 0)),
            pl.BlockSpec((ROWS, 1), lambda i: (i, 0)),
            pl.BlockSpec((ROWS, 1), lambda i: (i, 0)),
            pl.BlockSpec((8, 64), lambda i: (0, 0)),
            pl.BlockSpec((8, 64), lambda i: (0, 0)),
            pl.BlockSpec((8, 64), lambda i: (0, 0)),
        ],
        out_specs=pl.BlockSpec((ROWS, 3), lambda i: (i, 0)),
        out_shape=jax.ShapeDtypeStruct((N, 3), _f32),
    )(h, nx, ny, nz, xc, yc, zc, sums, aux, w2p)


def kernel(feat, coord, batch, W1, b1, gamma, beta, W2, b2):
    del b1
    coord = coord.astype(_f32)
    x = coord[:, 0]
    y = coord[:, 1]
    z = coord[:, 2]
    bi = batch.astype(jnp.int32)

    nx, ny, nz = _get_sc_knn()(x, y, z, bi)

    w1t = W1.astype(_f32).T
    h, sums = _tc_a(feat.astype(_f32), w1t, bi.reshape(N, 1))

    aux = jnp.zeros((8, 64), _f32)
    aux = aux.at[0, :].set(gamma.astype(_f32))
    aux = aux.at[1, :].set(beta.astype(_f32))
    aux = aux.at[2, :3].set(b2.astype(_f32))
    w2p = jnp.zeros((8, 64), _f32).at[:3, :].set(W2.astype(_f32))

    out = _tc_b(h, nx, ny, nz,
                x.reshape(N, 1), y.reshape(N, 1), z.reshape(N, 1),
                sums, aux, w2p)
    return out


# speedup vs baseline: 1.0804x; 1.0804x over previous
import functools

import jax
import jax.numpy as jnp
from jax import lax
from jax.experimental import pallas as pl
from jax.experimental.pallas import tpu as pltpu
from jax.experimental.pallas import tpu_sc as plsc

N = 16384
K = 16
BS = 1024
LANES = 16
NWORKERS = 32
QPW = N // NWORKERS
NBLK = N // LANES
FLUSH = 128
UNROLL = 8
P1_UNROLL = 8
GRIDC = ((0.05, 0.05, 0.05), (0.2, 0.2, 0.2), (0.1, 0.1, 2.0))

_f32 = jnp.float32
_INF = float("inf")


def _bf16r(v):
    u = plsc.bitcast(v, jnp.int32)
    r = u + 0x7FFF + ((u >> 16) & 1)
    r = r & jnp.int32(-65536)
    return plsc.bitcast(r, _f32)


def _sc_knn_body(x_hbm, y_hbm, z_hbm, b_hbm, ox_hbm, oy_hbm, oz_hbm,
                 xv, yv, zv, sxv, xbv, ybv, zbv, probev, bufx, bufy, bufz):
    cid = lax.axis_index("c")
    sid = lax.axis_index("s")
    wid = sid * 2 + cid
    pltpu.sync_copy(x_hbm, xv)
    pltpu.sync_copy(y_hbm, yv)
    pltpu.sync_copy(z_hbm, zv)

    lane_iota = lax.iota(jnp.int32, LANES)

    def _sx_step(b, _):
        s = b * LANES
        cx = xv[pl.ds(s, LANES)]
        cy = yv[pl.ds(s, LANES)]
        cz = zv[pl.ds(s, LANES)]
        sxv[pl.ds(s, LANES)] = cx * cx + cy * cy + cz * cz
        xbv[pl.ds(s, LANES)] = _bf16r(cx)
        ybv[pl.ds(s, LANES)] = _bf16r(cy)
        zbv[pl.ds(s, LANES)] = _bf16r(cz)
        return 0

    lax.fori_loop(0, NBLK, _sx_step, 0)

    def _bs_step(i, lo):
        step = 512 >> i
        nlo = lo + step
        addr = jnp.minimum(nlo, NBLK - 1) * LANES
        pltpu.sync_copy(b_hbm.at[pl.ds(addr, LANES)], probev)
        p0 = probev[...][0]
        return jnp.where((nlo < NBLK) & (p0 == 0), nlo, lo)

    lo_blk = lax.fori_loop(0, 10, _bs_step, jnp.zeros((), jnp.int32))
    pltpu.sync_copy(b_hbm.at[pl.ds(lo_blk * LANES, LANES)], probev)
    bvec = probev[...]
    nzeros = jnp.sum(jnp.where(bvec == 0, 1.0, 0.0)).astype(jnp.int32)
    count0 = lo_blk * LANES + nzeros

    qbase = wid * QPW

    def per_query(qi, k):
        q = qbase + qi
        in0 = q < count0
        seg_lo = jnp.where(in0, 0, count0)
        seg_hi = jnp.where(in0, count0, N)
        chunk = (q - seg_lo) // BS
        ex_lo = seg_lo + chunk * BS
        ex_hi = jnp.minimum(ex_lo + BS, seg_hi)
        lane = qi % LANES
        qblk = qbase + (qi - lane)
        sel = lane_iota == lane
        qsx = jnp.sum(jnp.where(sel, sxv[pl.ds(qblk, LANES)], 0.0))
        qxb = jnp.full((LANES,), jnp.sum(jnp.where(sel, xbv[pl.ds(qblk, LANES)], 0.0)), _f32)
        qyb = jnp.full((LANES,), jnp.sum(jnp.where(sel, ybv[pl.ds(qblk, LANES)], 0.0)), _f32)
        qzb = jnp.full((LANES,), jnp.sum(jnp.where(sel, zbv[pl.ds(qblk, LANES)], 0.0)), _f32)

        def block_key(base):
            prod = (xbv[pl.ds(base, LANES)] * qxb
                    + ybv[pl.ds(base, LANES)] * qyb
                    + zbv[pl.ds(base, LANES)] * qzb)
            t1 = qsx + sxv[pl.ds(base, LANES)]
            return jnp.maximum(t1 - 2.0 * prod, 1e-12)

        def merge16(args):
            td, ti, thr, d2c, b0 = args
            sd, si = plsc.sort_key_val(d2c, b0 + lane_iota)
            rsd = lax.rev(sd, (0,))
            rsi = lax.rev(si, (0,))
            keep = td <= rsd
            md = jnp.where(keep, td, rsd)
            mi = jnp.where(keep, ti, rsi)
            nd, ni = plsc.sort_key_val(md, mi)
            return nd, ni, jnp.minimum(thr, jnp.max(nd))

        def maybe_merge(top_d, top_i, thr, d2m, base):
            bmin = jnp.min(d2m)
            return lax.cond(bmin < thr, merge16,
                            lambda a: (a[0], a[1], a[2]),
                            (top_d, top_i, thr, d2m, base))

        def walk_range(lo, hi, carry, on_masked, on_group, on_block, unroll):
            def do(c):
                b_first = lo // LANES
                b_last = (hi - 1) // LANES
                c = on_masked(b_first, lo, hi, c)
                n_int = jnp.maximum(b_last - b_first - 1, 0)
                n_g = n_int // unroll
                b0 = b_first + 1
                c = lax.fori_loop(
                    0, n_g,
                    lambda g, c2: on_group((b0 + g * unroll) * LANES, c2), c)
                c = lax.fori_loop(
                    0, n_int - n_g * unroll,
                    lambda r, c2: on_block((b0 + n_g * unroll + r) * LANES,
                                           c2), c)
                return lax.cond(b_last > b_first,
                                lambda c2: on_masked(b_last, lo, hi, c2),
                                lambda c2: c2, c)

            return lax.cond(hi > lo, do, lambda c: c, carry)

        def p1_acc(c, d):
            m1, m2 = c
            return (jnp.minimum(m1, d),
                    jnp.minimum(m2, jnp.maximum(m1, d)))

        def p1_masked(b, lo, hi, c):
            base = b * LANES
            gidx = base + lane_iota
            ok = (gidx >= lo) & (gidx < hi)
            return p1_acc(c, jnp.where(ok, block_key(base), _INF))

        def p1_group(gb, c):
            for u in range(P1_UNROLL):
                c = p1_acc(c, block_key(gb + u * LANES))
            return c

        def p1_block(base, c):
            return p1_acc(c, block_key(base))

        m0 = (jnp.full((LANES,), _INF, _f32), jnp.full((LANES,), _INF, _f32))
        m0 = walk_range(seg_lo, ex_lo, m0, p1_masked, p1_group, p1_block,
                        P1_UNROLL)
        m1, m2 = walk_range(ex_hi, seg_hi, m0, p1_masked, p1_group, p1_block,
                            P1_UNROLL)
        s1, _ = plsc.sort_key_val(m1, lane_iota)
        s2, _ = plsc.sort_key_val(m2, lane_iota)
        low16 = jnp.minimum(s1, lax.rev(s2, (0,)))
        thr0 = jnp.max(low16) * (1.0 + 3e-7)

        def p2_masked(b, lo, hi, carry):
            top_d, top_i, thr = carry
            base = b * LANES
            gidx = base + lane_iota
            ok = (gidx >= lo) & (gidx < hi)
            d2m = jnp.where(ok, block_key(base), _INF)
            return maybe_merge(top_d, top_i, thr, d2m, base)

        def p2_group(gb, c2):
            td, ti, thr = c2
            thrv = jnp.full((LANES,), thr, _f32)
            ds = [block_key(gb + u * LANES) for u in range(UNROLL)]
            bits = None
            for u in range(UNROLL):
                nh = plsc.all_reduce_population_count(ds[u] < thrv)
                bu = jnp.where(nh > 0, jnp.int32(1 << u), jnp.int32(0))
                bits = bu if bits is None else bits | bu
            bitsc = bits[0]

            def hit(args):
                t = args
                for u in range(UNROLL):
                    t = lax.cond(
                        ((bitsc >> u) & 1) != 0, merge16,
                        lambda a: (a[0], a[1], a[2]),
                        (*t, ds[u], gb + u * LANES))
                return t

            return lax.cond(bitsc != 0, hit, lambda a: a, (td, ti, thr))

        def p2_block(base, c2):
            td, ti, thr = c2
            return maybe_merge(td, ti, thr, block_key(base), base)

        top = (jnp.full((LANES,), _INF, _f32), lane_iota, thr0)
        top = walk_range(seg_lo, ex_lo, top, p2_masked, p2_group, p2_block,
                         UNROLL)
        top_d, top_i, _ = walk_range(ex_hi, seg_hi, top, p2_masked, p2_group,
                                     p2_block, UNROLL)

        bufx[k, :] = plsc.load_gather(xv, [top_i])
        bufy[k, :] = plsc.load_gather(yv, [top_i])
        bufz[k, :] = plsc.load_gather(zv, [top_i])
        return 0

    def super_group(sg, _):
        def pq(k, _2):
            return per_query(sg * FLUSH + k, k)

        lax.fori_loop(0, FLUSH, pq, 0)
        row0 = qbase + sg * FLUSH
        pltpu.sync_copy(bufx, ox_hbm.at[pl.ds(row0, FLUSH), :])
        pltpu.sync_copy(bufy, oy_hbm.at[pl.ds(row0, FLUSH), :])
        pltpu.sync_copy(bufz, oz_hbm.at[pl.ds(row0, FLUSH), :])
        return 0

    lax.fori_loop(0, QPW // FLUSH, super_group, 0)


@functools.lru_cache(maxsize=1)
def _get_sc_knn():
    return pl.kernel(
        _sc_knn_body,
        out_type=(jax.ShapeDtypeStruct((N, K), _f32),
                  jax.ShapeDtypeStruct((N, K), _f32),
                  jax.ShapeDtypeStruct((N, K), _f32)),
        mesh=plsc.VectorSubcoreMesh(core_axis_name="c", subcore_axis_name="s"),
        scratch_types=[
            pltpu.VMEM((N,), _f32),
            pltpu.VMEM((N,), _f32),
            pltpu.VMEM((N,), _f32),
            pltpu.VMEM((N,), _f32),
            pltpu.VMEM((N,), _f32),
            pltpu.VMEM((N,), _f32),
            pltpu.VMEM((N,), _f32),
            pltpu.VMEM((LANES,), jnp.int32),
            pltpu.VMEM((FLUSH, K), _f32),
            pltpu.VMEM((FLUSH, K), _f32),
            pltpu.VMEM((FLUSH, K), _f32),
        ],
        compiler_params=pltpu.CompilerParams(needs_layout_passes=False,
                                             use_tc_tiling_on_sc=False),
    )


ROWS = 1024
NROWBLK = N // ROWS


def _tc_a_body(feat_ref, w1t_ref, b_ref, h_ref, sums_ref):
    pid = pl.program_id(0)
    h = jnp.dot(feat_ref[...], w1t_ref[...], preferred_element_type=_f32)
    h_ref[...] = h
    m0 = (b_ref[...] == 0).astype(_f32)
    hm0 = h * m0
    hm1 = h - hm0
    s0 = jnp.sum(hm0, axis=0, keepdims=True)
    s0q = jnp.sum(h * hm0, axis=0, keepdims=True)
    s1 = jnp.sum(hm1, axis=0, keepdims=True)
    s1q = jnp.sum(h * hm1, axis=0, keepdims=True)
    c0 = jnp.sum(m0)
    c0row = jnp.full((1, 64), c0, _f32)
    contrib = jnp.concatenate(
        [s0, s0q, s1, s1q, c0row, jnp.zeros((3, 64), _f32)], axis=0)

    @pl.when(pid == 0)
    def _():
        sums_ref[...] = jnp.zeros_like(sums_ref)

    sums_ref[...] += contrib


def _tc_a(feat, w1t, b2d):
    return pl.pallas_call(
        _tc_a_body,
        grid=(NROWBLK,),
        in_specs=[
            pl.BlockSpec((ROWS, 128), lambda i: (i, 0)),
            pl.BlockSpec((128, 64), lambda i: (0, 0)),
            pl.BlockSpec((ROWS, 1), lambda i: (i, 0)),
        ],
        out_specs=[
            pl.BlockSpec((ROWS, 64), lambda i: (i, 0)),
            pl.BlockSpec((8, 64), lambda i: (0, 0)),
        ],
        out_shape=[
            jax.ShapeDtypeStruct((N, 64), _f32),
            jax.ShapeDtypeStruct((8, 64), _f32),
        ],
    )(feat, w1t, b2d)


def _tc_b_body(h_ref, nx_ref, ny_ref, nz_ref, xc_ref, yc_ref, zc_ref,
               sums_ref, aux_ref, w2p_ref, out_ref):
    pid = pl.program_id(0)
    kf = _f32(K)
    nx = nx_ref[...]
    ny = ny_ref[...]
    nz = nz_ref[...]
    xc = xc_ref[...]
    yc = yc_ref[...]
    zc = zc_ref[...]
    dxn = nx - xc
    dyn = ny - yc
    dzn = nz - zc
    def _bf(v):
        return v.astype(jnp.bfloat16).astype(_f32)

    sq = xc * xc + yc * yc + zc * zc
    sn = nx * nx + ny * ny + nz * nz
    prod = _bf(nx) * _bf(xc) + _bf(ny) * _bf(yc) + _bf(nz) * _bf(zc)
    nd2 = (sq + sn) - 2.0 * prod
    nd = jnp.sqrt(jnp.maximum(nd2, 1e-12))
    mean_dist = jnp.mean(nd, axis=1, keepdims=True)
    density = 1.0 / (mean_dist + 1e-6)

    mx = jnp.mean(dxn, axis=1, keepdims=True)
    my = jnp.mean(dyn, axis=1, keepdims=True)
    mz = jnp.mean(dzn, axis=1, keepdims=True)
    inv = _f32(1.0 / (K - 1))
    cxx = (jnp.sum(dxn * dxn, 1, keepdims=True) - kf * mx * mx) * inv
    cyy = (jnp.sum(dyn * dyn, 1, keepdims=True) - kf * my * my) * inv
    czz = (jnp.sum(dzn * dzn, 1, keepdims=True) - kf * mz * mz) * inv
    cxy = (jnp.sum(dxn * dyn, 1, keepdims=True) - kf * mx * my) * inv
    cxz = (jnp.sum(dxn * dzn, 1, keepdims=True) - kf * mx * mz) * inv
    cyz = (jnp.sum(dyn * dzn, 1, keepdims=True) - kf * my * mz) * inv
    q3 = (cxx + cyy + czz) * _f32(1.0 / 3.0)
    bxx = cxx - q3
    byy = cyy - q3
    bzz = czz - q3
    p2 = ((bxx * bxx + byy * byy + bzz * bzz) * _f32(1.0 / 6.0)
          + (cxy * cxy + cxz * cxz + cyz * cyz) * _f32(1.0 / 3.0))
    detb = (bxx * (byy * bzz - cyz * cyz) - cxy * (cxy * bzz - cyz * cxz)
            + cxz * (cxy * cyz - byy * cxz))
    p = jnp.sqrt(p2)
    mu = 2.0 * p
    for _ in range(40):
        g = mu * mu * mu - 3.0 * p2 * mu - detb
        gp = 3.0 * mu * mu - 3.0 * p2
        mu = mu - g / (gp + 1e-30)
    mu = jnp.where(p2 > 1e-30, mu, 0.0)
    linearity = (2.0 * mu - q3) / (3.0 * q3 + 1e-6)

    c0 = sums_ref[4, 0]
    c1 = _f32(N) - c0
    mu0 = sums_ref[0:1, :] / c0
    mu1 = sums_ref[2:3, :] / c1
    v0 = jnp.maximum(sums_ref[1:2, :] / c0 - mu0 * mu0, 0.0)
    v1 = jnp.maximum(sums_ref[3:4, :] / c1 - mu1 * mu1, 0.0)
    rows = lax.broadcasted_iota(jnp.int32, (ROWS, 1), 0) + pid * ROWS
    in0 = rows < c0.astype(jnp.int32)
    h = h_ref[...]
    mean = jnp.where(in0, mu0, mu1)
    var = jnp.where(in0, v0, v1)
    gamma = aux_ref[0:1, :]
    beta = aux_ref[1:2, :]
    hn = (h - mean) / jnp.sqrt(var + 1e-5) * gamma + beta
    h2 = jnp.maximum(hn, 0.0)
    h2b = _bf(h2)
    l0 = jnp.sum(h2b * _bf(w2p_ref[0:1, :]), axis=1, keepdims=True) + aux_ref[2, 0]
    l1 = jnp.sum(h2b * _bf(w2p_ref[1:2, :]), axis=1, keepdims=True) + aux_ref[2, 1]
    l2 = jnp.sum(h2b * _bf(w2p_ref[2:3, :]), axis=1, keepdims=True) + aux_ref[2, 2]
    lm = jnp.maximum(l0, jnp.maximum(l1, l2))
    e0 = jnp.exp(l0 - lm)
    e1 = jnp.exp(l1 - lm)
    e2 = jnp.exp(l2 - lm)
    zs = e0 + e1 + e2
    p0 = e0 / zs
    p1 = e1 / zs
    p2_ = e2 / zs

    tower = (density * 2.0 + p0) * _f32(1.0 / 3.0)
    background = (jnp.maximum(1.0 - linearity, 1.0 - density) + p1) * _f32(1.0 / 3.0)
    line = (linearity * 2.0 + p2_) * _f32(1.0 / 3.0)
    cols = []
    for a in range(3):
        v = (tower * GRIDC[0][a] + background * GRIDC[1][a]
             + line * GRIDC[2][a] + 1e-6)
        cols.append(v)
    computed = jnp.concatenate(cols, axis=1)
    cnt = jnp.where(in0, c0, c1)
    fallback = jnp.concatenate(
        [jnp.full((ROWS, 1), GRIDC[1][a], _f32) for a in range(3)], axis=1)
    out_ref[...] = jnp.where(cnt >= kf, computed, fallback)


def _tc_b(h, nx, ny, nz, xc, yc, zc, sums, aux, w2p):
    return pl.pallas_call(
        _tc_b_body,
        grid=(NROWBLK,),
        in_specs=[
            pl.BlockSpec((ROWS, 64), lambda i: (i, 0)),
            pl.BlockSpec((ROWS, K), lambda i: (i, 0)),
            pl.BlockSpec((ROWS, K), lambda i: (i, 0)),
            pl.BlockSpec((ROWS, K), lambda i: (i, 0)),
            pl.BlockSpec((ROWS, 1), lambda i: (i, 0)),
            pl.BlockSpec((ROWS, 1), lambda i: (i, 0)),
            pl.BlockSpec((ROWS, 1), lambda i: (i, 0)),
            pl.BlockSpec((8, 64), lambda i: (0, 0)),
            pl.BlockSpec((8, 64), lambda i: (0, 0)),
            pl.BlockSpec((8, 64), lambda i: (0, 0)),
        ],
        out_specs=pl.BlockSpec((ROWS, 3), lambda i: (i, 0)),
        out_shape=jax.ShapeDtypeStruct((N, 3), _f32),
    )(h, nx, ny, nz, xc, yc, zc, sums, aux, w2p)


def kernel(feat, coord, batch, W1, b1, gamma, beta, W2, b2):
    del b1
    coord = coord.astype(_f32)
    x = coord[:, 0]
    y = coord[:, 1]
    z = coord[:, 2]
    bi = batch.astype(jnp.int32)

    nx, ny, nz = _get_sc_knn()(x, y, z, bi)

    w1t = W1.astype(_f32).T
    h, sums = _tc_a(feat.astype(_f32), w1t, bi.reshape(N, 1))

    aux = jnp.zeros((8, 64), _f32)
    aux = aux.at[0, :].set(gamma.astype(_f32))
    aux = aux.at[1, :].set(beta.astype(_f32))
    aux = aux.at[2, :3].set(b2.astype(_f32))
    w2p = jnp.zeros((8, 64), _f32).at[:3, :].set(W2.astype(_f32))

    out = _tc_b(h, nx, ny, nz,
                x.reshape(N, 1), y.reshape(N, 1), z.reshape(N, 1),
                sums, aux, w2p)
    return out
